# baseline (speedup 1.0000x reference)
import jax
import jax.numpy as jnp
from jax import lax
from jax.experimental import pallas as pl
from jax.experimental.pallas import tpu as pltpu
from jax.experimental.pallas import tpu_sc as plsc

N_ROWS = 1000000
D_FEAT = 64
BATCH = 16384

NC = 2
NS = 16
NW = NC * NS
B_PER_W = BATCH // NW
NLIST = 4
LIST_W = B_PER_W // NLIST


def _gather_kernel(table_hbm, resp_hbm, idx_hbm, out_rows_hbm, out_resp_hbm,
                   idx_v, rows_v, resp_v, sem_rows, sem_resp):
    wid = lax.axis_index("s") * NC + lax.axis_index("c")
    base = wid * B_PER_W

    pltpu.sync_copy(idx_hbm.at[pl.ds(base, B_PER_W)], idx_v)

    for j in range(NLIST):
        sub = idx_v.at[pl.ds(j * LIST_W, LIST_W)]
        pltpu.async_copy(table_hbm.at[sub],
                         rows_v.at[pl.ds(j * LIST_W, LIST_W)], sem_rows)
        pltpu.async_copy(resp_hbm.at[sub],
                         resp_v.at[pl.ds(j * LIST_W, LIST_W)], sem_resp)

    pltpu.make_async_copy(out_rows_hbm.at[pl.ds(0, B_PER_W)], rows_v,
                          sem_rows).wait()
    pltpu.make_async_copy(resp_hbm.at[pl.ds(0, B_PER_W)], resp_v,
                          sem_resp).wait()

    pltpu.sync_copy(rows_v, out_rows_hbm.at[pl.ds(base, B_PER_W)])
    pltpu.sync_copy(resp_v, out_resp_hbm.at[pl.ds(base, B_PER_W)])


def kernel(input, response, mb_idx):
    mesh = plsc.VectorSubcoreMesh(core_axis_name="c", subcore_axis_name="s")
    out_rows, out_resp = pl.kernel(
        _gather_kernel,
        mesh=mesh,
        compiler_params=pltpu.CompilerParams(use_tc_tiling_on_sc=False),
        out_type=(
            jax.ShapeDtypeStruct((BATCH, D_FEAT), jnp.float32),
            jax.ShapeDtypeStruct((BATCH,), jnp.float32),
        ),
        scratch_types=[
            pltpu.VMEM((B_PER_W,), jnp.int32),
            pltpu.VMEM((B_PER_W, D_FEAT), jnp.float32),
            pltpu.VMEM((B_PER_W,), jnp.float32),
            pltpu.SemaphoreType.DMA,
            pltpu.SemaphoreType.DMA,
        ],
    )(input, response, mb_idx)
    return out_rows, out_resp


# speedup vs baseline: 2.7241x; 2.7241x over previous
import jax
import jax.numpy as jnp
from jax import lax
from jax.experimental import pallas as pl
from jax.experimental.pallas import tpu as pltpu
from jax.experimental.pallas import tpu_sc as plsc

N_ROWS = 1000000
D_FEAT = 64
BATCH = 16384

NC = 2
NS = 16
NW = NC * NS
B_PER_W = BATCH // NW
NG = B_PER_W // 16
NVREG = BATCH // 16

CHUNK = 512
N_CHUNKS = (N_ROWS + CHUNK - 1) // CHUNK
PHYS_MINOR = ((N_ROWS + 127) // 128) * 128
MAX_LANE_LO = ((PHYS_MINOR - CHUNK) // 128) * 128
CH_BASE = N_CHUNKS // NW
CH_EXTRA = N_CHUNKS - CH_BASE * NW
RING = 32


def _gather_kernel(tableT_hbm, resp_hbm, idx_hbm, out_rows_hbm, out_resp_hbm,
                   idx_all, hist_v, csr_v, cur_v, hits_r, hits_k,
                   chunk_v, rowbuf, resp16_v, resp_v, sem_rows, sem_resp):
    wid = lax.axis_index("s") * NC + lax.axis_index("c")
    base = wid * B_PER_W
    lanes = lax.iota(jnp.int32, 16)
    ones = jnp.ones((16,), jnp.int32)

    pltpu.sync_copy(idx_hbm.at[pl.ds(0, BATCH)], idx_all)

    def fire_resp(g, _):
        vec = idx_all[pl.ds(pl.multiple_of(base + g * 16, 8), 16)]
        for j in range(16):
            k = g * 16 + j
            r16 = pl.multiple_of(
                lax.shift_left(lax.shift_right_logical(vec[j], 4), 4), 8)
            pltpu.async_copy(resp_hbm.at[pl.ds(r16, 16)],
                             resp16_v.at[pl.ds(pl.multiple_of(k * 16, 8), 16)],
                             sem_resp)
        return _

    lax.fori_loop(0, NG, fire_resp, None)

    n_c = CH_BASE + jnp.where(wid < CH_EXTRA, 1, 0)
    base_c = wid * CH_BASE + jnp.minimum(wid, CH_EXTRA)
    lo_r = base_c * CHUNK
    hi_r = (base_c + n_c) * CHUNK

    for i in range(4):
        hist_v[pl.ds(i * 16, 16)] = jnp.zeros((16,), jnp.int32)

    def hist(v, _):
        vec = idx_all[pl.ds(pl.multiple_of(v * 16, 8), 16)]
        m = jnp.logical_and(vec >= lo_r, vec < hi_r)
        crel = lax.shift_right_logical(vec - lo_r, 9)
        plsc.addupdate_scatter(hist_v, [crel], ones, mask=m)
        return _

    lax.fori_loop(0, NVREG, hist, None)

    carry = jnp.zeros((16,), jnp.int32)
    for i in range(4):
        h = hist_v[pl.ds(i * 16, 16)]
        inc = plsc.cumsum(h)
        csr_v[pl.ds(i * 16, 16)] = inc - h + carry
        cur_v[pl.ds(i * 16, 16)] = inc - h + carry
        tot = jnp.take_along_axis(
            inc, jnp.full((16,), 15, jnp.int32), axis=0,
            mode=lax.GatherScatterMode.PROMISE_IN_BOUNDS)
        carry = carry + tot
    csr_v[pl.ds(64, 16)] = carry

    def place(v, _):
        vec = idx_all[pl.ds(pl.multiple_of(v * 16, 8), 16)]
        m = jnp.logical_and(vec >= lo_r, vec < hi_r)
        crel = lax.shift_right_logical(vec - lo_r, 9)
        cnt = plsc.all_reduce_population_count(m)[0]

        def one_hit(_, mb):
            lane = plsc.all_reduce_ffs(mb)
            crel_j = jnp.take_along_axis(
                crel, lane, axis=0,
                mode=lax.GatherScatterMode.PROMISE_IN_BOUNDS)
            r_j = jnp.take_along_axis(
                vec, lane, axis=0,
                mode=lax.GatherScatterMode.PROMISE_IN_BOUNDS)
            slot = plsc.load_gather(cur_v, [crel_j])
            lane0 = lanes == 0
            plsc.store_scatter(hits_r, [slot], r_j, mask=lane0)
            plsc.store_scatter(hits_k, [slot], v * 16 + lane, mask=lane0)
            plsc.addupdate_scatter(cur_v, [crel_j], ones, mask=lane0)
            return jnp.logical_and(mb, lanes != lane)

        lax.fori_loop(0, cnt, one_hit, m)
        return _

    lax.fori_loop(0, NVREG, place, None)

    def do_chunk(c, _):
        lane_lo = jnp.minimum((base_c + c) * CHUNK, MAX_LANE_LO)
        lane_lo = pl.multiple_of(lane_lo, 128)
        pltpu.sync_copy(tableT_hbm.at[:, pl.ds(lane_lo, CHUNK)], chunk_v)
        cvec = jnp.full((16,), c, jnp.int32)
        s = plsc.load_gather(csr_v, [cvec])[0]
        e = plsc.load_gather(csr_v, [cvec + 1])[0]

        def batch(b, _):
            b0 = s + b * RING
            nb = jnp.minimum(e - b0, RING)

            def one(i, _):
                hvec = jnp.full((16,), b0 + i, jnp.int32)
                r = plsc.load_gather(hits_r, [hvec])[0]
                k = plsc.load_gather(hits_k, [hvec])[0]
                rl = jnp.full((16,), r - lane_lo, jnp.int32)
                for fb in range(4):
                    col = plsc.load_gather(chunk_v, [fb * 16 + lanes, rl])
                    rowbuf[i, pl.ds(fb * 16, 16)] = col
                pltpu.async_copy(rowbuf.at[pl.ds(i, 1)],
                                 out_rows_hbm.at[pl.ds(k, 1)], sem_rows)
                return _

            lax.fori_loop(0, nb, one, None)

            def drain(i, _):
                pltpu.make_async_copy(rowbuf.at[pl.ds(0, 1)],
                                      out_rows_hbm.at[pl.ds(0, 1)],
                                      sem_rows).wait()
                return _

            lax.fori_loop(0, nb, drain, None)
            return _

        n_batches = lax.div(e - s + (RING - 1), RING)
        lax.fori_loop(0, n_batches, batch, None)
        return _

    lax.fori_loop(0, n_c, do_chunk, None)

    pltpu.make_async_copy(resp_hbm.at[pl.ds(0, 16 * B_PER_W)], resp16_v,
                          sem_resp).wait()

    def pick(g, _):
        vec = idx_all[pl.ds(pl.multiple_of(base + g * 16, 8), 16)]
        sub = lax.bitwise_and(vec, 15)
        flat = (g * 16 + lanes) * 16 + sub
        resp_v[pl.ds(pl.multiple_of(g * 16, 8), 16)] = (
            plsc.load_gather(resp16_v, [flat]))
        return _

    lax.fori_loop(0, NG, pick, None)
    pltpu.sync_copy(resp_v, out_resp_hbm.at[pl.ds(base, B_PER_W)])


def kernel(input, response, mb_idx):
    mesh = plsc.VectorSubcoreMesh(core_axis_name="c", subcore_axis_name="s")
    out_rows, out_resp = pl.kernel(
        _gather_kernel,
        mesh=mesh,
        compiler_params=pltpu.CompilerParams(needs_layout_passes=False),
        out_type=(
            jax.ShapeDtypeStruct((BATCH, D_FEAT), jnp.float32),
            jax.ShapeDtypeStruct((BATCH,), jnp.float32),
        ),
        scratch_types=[
            pltpu.VMEM((BATCH,), jnp.int32),
            pltpu.VMEM((80,), jnp.int32),
            pltpu.VMEM((80,), jnp.int32),
            pltpu.VMEM((80,), jnp.int32),
            pltpu.VMEM((BATCH,), jnp.int32),
            pltpu.VMEM((BATCH,), jnp.int32),
            pltpu.VMEM((D_FEAT, CHUNK), jnp.float32),
            pltpu.VMEM((RING, D_FEAT), jnp.float32),
            pltpu.VMEM((16 * B_PER_W,), jnp.float32),
            pltpu.VMEM((B_PER_W,), jnp.float32),
            pltpu.SemaphoreType.DMA,
            pltpu.SemaphoreType.DMA,
        ],
    )(input.T, response, mb_idx)
    return out_rows, out_resp


# speedup vs baseline: 3.4409x; 1.2632x over previous
import jax
import jax.numpy as jnp
from jax import lax
from jax.experimental import pallas as pl
from jax.experimental.pallas import tpu as pltpu
from jax.experimental.pallas import tpu_sc as plsc

N_ROWS = 1000000
D_FEAT = 64
BATCH = 16384

NC = 2
NS = 16
NW = NC * NS
B_PER_W = BATCH // NW
NG = B_PER_W // 16
NVREG = BATCH // 16

CHUNK = 512
N_CHUNKS = (N_ROWS + CHUNK - 1) // CHUNK
PHYS_MINOR = ((N_ROWS + 127) // 128) * 128
MAX_LANE_LO = ((PHYS_MINOR - CHUNK) // 128) * 128
CH_BASE = N_CHUNKS // NW
CH_EXTRA = N_CHUNKS - CH_BASE * NW
RING = 32


def _gather_kernel(tableT_hbm, resp_hbm, idx_hbm, out_rows_hbm, out_resp_hbm,
                   idx_all, hist_v, csr_v, cur_v, hits_r, hits_k,
                   chunk_v, rowbuf, resp16_v, resp_v,
                   sem_rows, sem_resp, sem_ch0, sem_ch1):
    wid = lax.axis_index("s") * NC + lax.axis_index("c")
    base = wid * B_PER_W
    lanes = lax.iota(jnp.int32, 16)
    ones = jnp.ones((16,), jnp.int32)

    pltpu.sync_copy(idx_hbm.at[pl.ds(0, BATCH)], idx_all)

    def fire_resp(g, _):
        vec = idx_all[pl.ds(pl.multiple_of(base + g * 16, 8), 16)]
        for j in range(16):
            k = g * 16 + j
            r16 = pl.multiple_of(
                lax.shift_left(lax.shift_right_logical(vec[j], 4), 4), 8)
            pltpu.async_copy(resp_hbm.at[pl.ds(r16, 16)],
                             resp16_v.at[pl.ds(pl.multiple_of(k * 16, 8), 16)],
                             sem_resp)
        return _

    lax.fori_loop(0, NG, fire_resp, None)

    n_c = CH_BASE + jnp.where(wid < CH_EXTRA, 1, 0)
    base_c = wid * CH_BASE + jnp.minimum(wid, CH_EXTRA)
    lo_r = base_c * CHUNK
    hi_r = (base_c + n_c) * CHUNK

    for i in range(4):
        hist_v[pl.ds(i * 16, 16)] = jnp.zeros((16,), jnp.int32)

    def hist(v, _):
        vec = idx_all[pl.ds(pl.multiple_of(v * 16, 8), 16)]
        m = jnp.logical_and(vec >= lo_r, vec < hi_r)
        crel = lax.shift_right_logical(vec - lo_r, 9)
        plsc.addupdate_scatter(hist_v, [crel], ones, mask=m)
        return _

    lax.fori_loop(0, NVREG, hist, None)

    carry = jnp.zeros((16,), jnp.int32)
    for i in range(4):
        h = hist_v[pl.ds(i * 16, 16)]
        inc = plsc.cumsum(h)
        csr_v[pl.ds(i * 16, 16)] = inc - h + carry
        cur_v[pl.ds(i * 16, 16)] = inc - h + carry
        tot = jnp.take_along_axis(
            inc, jnp.full((16,), 15, jnp.int32), axis=0,
            mode=lax.GatherScatterMode.PROMISE_IN_BOUNDS)
        carry = carry + tot
    csr_v[pl.ds(64, 16)] = carry

    def place(v, _):
        vec = idx_all[pl.ds(pl.multiple_of(v * 16, 8), 16)]
        m = jnp.logical_and(vec >= lo_r, vec < hi_r)
        crel = lax.shift_right_logical(vec - lo_r, 9)
        cnt = plsc.all_reduce_population_count(m)[0]

        def one_hit(_, mb):
            lane = plsc.all_reduce_ffs(mb)
            crel_j = jnp.take_along_axis(
                crel, lane, axis=0,
                mode=lax.GatherScatterMode.PROMISE_IN_BOUNDS)
            r_j = jnp.take_along_axis(
                vec, lane, axis=0,
                mode=lax.GatherScatterMode.PROMISE_IN_BOUNDS)
            slot = plsc.load_gather(cur_v, [crel_j])
            lane0 = lanes == 0
            plsc.store_scatter(hits_r, [slot], r_j, mask=lane0)
            plsc.store_scatter(hits_k, [slot], v * 16 + lane, mask=lane0)
            plsc.addupdate_scatter(cur_v, [crel_j], ones, mask=lane0)
            return jnp.logical_and(mb, lanes != lane)

        lax.fori_loop(0, cnt, one_hit, m)
        return _

    lax.fori_loop(0, NVREG, place, None)

    def window(c):
        lo = jnp.minimum((base_c + c) * CHUNK, MAX_LANE_LO)
        return pl.multiple_of(lo, 128)

    sem_chunk = (sem_ch0, sem_ch1)
    for b in range(2):
        @pl.when(b < n_c)
        def _prime():
            pltpu.async_copy(tableT_hbm.at[:, pl.ds(window(b), CHUNK)],
                             chunk_v.at[b], sem_chunk[b])

    def process_chunk(c, bufi, lane_lo):
        cvec = jnp.full((16,), c, jnp.int32)
        bvec = jnp.full((16,), bufi, jnp.int32)
        s = plsc.load_gather(csr_v, [cvec])[0]
        e = plsc.load_gather(csr_v, [cvec + 1])[0]

        def batch(t, _):
            b0 = s + t * RING
            nb = jnp.minimum(e - b0, RING)

            def one(i, _):
                hvec = jnp.full((16,), b0 + i, jnp.int32)
                r = plsc.load_gather(hits_r, [hvec])[0]
                k = plsc.load_gather(hits_k, [hvec])[0]
                rl = jnp.full((16,), r - lane_lo, jnp.int32)
                for fb in range(4):
                    col = plsc.load_gather(
                        chunk_v, [bvec, fb * 16 + lanes, rl])
                    rowbuf[i, pl.ds(fb * 16, 16)] = col
                pltpu.async_copy(rowbuf.at[pl.ds(i, 1)],
                                 out_rows_hbm.at[pl.ds(k, 1)], sem_rows)
                return _

            lax.fori_loop(0, nb, one, None)

            def drain(i, _):
                pltpu.make_async_copy(rowbuf.at[pl.ds(0, 1)],
                                      out_rows_hbm.at[pl.ds(0, 1)],
                                      sem_rows).wait()
                return _

            lax.fori_loop(0, nb, drain, None)
            return _

        n_batches = lax.div(e - s + (RING - 1), RING)
        lax.fori_loop(0, n_batches, batch, None)

    def pair(p, _):
        for b in range(2):
            c = p * 2 + b

            @pl.when(c < n_c)
            def _do():
                pltpu.make_async_copy(
                    tableT_hbm.at[:, pl.ds(0, CHUNK)], chunk_v.at[b],
                    sem_chunk[b]).wait()
                process_chunk(c, b, window(c))

                @pl.when(c + 2 < n_c)
                def _prefetch():
                    pltpu.async_copy(
                        tableT_hbm.at[:, pl.ds(window(c + 2), CHUNK)],
                        chunk_v.at[b], sem_chunk[b])
        return _

    lax.fori_loop(0, lax.div(n_c + 1, 2), pair, None)

    pltpu.make_async_copy(resp_hbm.at[pl.ds(0, 16 * B_PER_W)], resp16_v,
                          sem_resp).wait()

    def pick(g, _):
        vec = idx_all[pl.ds(pl.multiple_of(base + g * 16, 8), 16)]
        sub = lax.bitwise_and(vec, 15)
        flat = (g * 16 + lanes) * 16 + sub
        resp_v[pl.ds(pl.multiple_of(g * 16, 8), 16)] = (
            plsc.load_gather(resp16_v, [flat]))
        return _

    lax.fori_loop(0, NG, pick, None)
    pltpu.sync_copy(resp_v, out_resp_hbm.at[pl.ds(base, B_PER_W)])


def kernel(input, response, mb_idx):
    mesh = plsc.VectorSubcoreMesh(core_axis_name="c", subcore_axis_name="s")
    out_rows, out_resp = pl.kernel(
        _gather_kernel,
        mesh=mesh,
        compiler_params=pltpu.CompilerParams(needs_layout_passes=False),
        out_type=(
            jax.ShapeDtypeStruct((BATCH, D_FEAT), jnp.float32),
            jax.ShapeDtypeStruct((BATCH,), jnp.float32),
        ),
        scratch_types=[
            pltpu.VMEM((BATCH,), jnp.int32),
            pltpu.VMEM((80,), jnp.int32),
            pltpu.VMEM((80,), jnp.int32),
            pltpu.VMEM((80,), jnp.int32),
            pltpu.VMEM((BATCH,), jnp.int32),
            pltpu.VMEM((BATCH,), jnp.int32),
            pltpu.VMEM((2, D_FEAT, CHUNK), jnp.float32),
            pltpu.VMEM((RING, D_FEAT), jnp.float32),
            pltpu.VMEM((16 * B_PER_W,), jnp.float32),
            pltpu.VMEM((B_PER_W,), jnp.float32),
            pltpu.SemaphoreType.DMA,
            pltpu.SemaphoreType.DMA,
            pltpu.SemaphoreType.DMA,
            pltpu.SemaphoreType.DMA,
        ],
    )(input.T, response, mb_idx)
    return out_rows, out_resp


# speedup vs baseline: 3.8220x; 1.1108x over previous
import jax
import jax.numpy as jnp
from jax import lax
from jax.experimental import pallas as pl
from jax.experimental.pallas import tpu as pltpu
from jax.experimental.pallas import tpu_sc as plsc

N_ROWS = 1000000
D_FEAT = 64
BATCH = 16384

NC = 2
NS = 16
NW = NC * NS
B_PER_W = BATCH // NW
NG = B_PER_W // 16
NVREG = BATCH // 16

NBUF = 4
CHUNK = 256
CHUNK_SHIFT = 8
N_CHUNKS = (N_ROWS + CHUNK - 1) // CHUNK
PHYS_MINOR = ((N_ROWS + 127) // 128) * 128
MAX_LANE_LO = ((PHYS_MINOR - CHUNK) // 128) * 128
CH_BASE = N_CHUNKS // NW
CH_EXTRA = N_CHUNKS - CH_BASE * NW
RING = 32


def _gather_kernel(tableT_hbm, resp_hbm, idx_hbm, out_rows_hbm, out_resp_hbm,
                   idx_all, hist_v, csr_v, cur_v, hits_r, hits_k,
                   chunk_v, rowbuf, resp16_v, resp_v,
                   sem_rows, sem_resp, sem_ch0, sem_ch1, sem_ch2, sem_ch3):
    wid = lax.axis_index("s") * NC + lax.axis_index("c")
    base = wid * B_PER_W
    lanes = lax.iota(jnp.int32, 16)
    ones = jnp.ones((16,), jnp.int32)

    pltpu.sync_copy(idx_hbm.at[pl.ds(0, BATCH)], idx_all)

    def fire_resp(g, _):
        vec = idx_all[pl.ds(pl.multiple_of(base + g * 16, 8), 16)]
        for j in range(16):
            k = g * 16 + j
            r16 = pl.multiple_of(
                lax.shift_left(lax.shift_right_logical(vec[j], 4), 4), 8)
            pltpu.async_copy(resp_hbm.at[pl.ds(r16, 16)],
                             resp16_v.at[pl.ds(pl.multiple_of(k * 16, 8), 16)],
                             sem_resp)
        return _

    lax.fori_loop(0, NG, fire_resp, None)

    n_c = CH_BASE + jnp.where(wid < CH_EXTRA, 1, 0)
    base_c = wid * CH_BASE + jnp.minimum(wid, CH_EXTRA)
    lo_r = base_c * CHUNK
    hi_r = (base_c + n_c) * CHUNK

    for i in range(8):
        hist_v[pl.ds(i * 16, 16)] = jnp.zeros((16,), jnp.int32)

    def hist(v, _):
        vec = idx_all[pl.ds(pl.multiple_of(v * 16, 8), 16)]
        m = jnp.logical_and(vec >= lo_r, vec < hi_r)
        crel = lax.shift_right_logical(vec - lo_r, CHUNK_SHIFT)
        plsc.addupdate_scatter(hist_v, [crel], ones, mask=m)
        return _

    lax.fori_loop(0, NVREG, hist, None)

    carry = jnp.zeros((16,), jnp.int32)
    for i in range(8):
        h = hist_v[pl.ds(i * 16, 16)]
        inc = plsc.cumsum(h)
        csr_v[pl.ds(i * 16, 16)] = inc - h + carry
        cur_v[pl.ds(i * 16, 16)] = inc - h + carry
        tot = jnp.take_along_axis(
            inc, jnp.full((16,), 15, jnp.int32), axis=0,
            mode=lax.GatherScatterMode.PROMISE_IN_BOUNDS)
        carry = carry + tot
    csr_v[pl.ds(128, 16)] = carry

    def place(v, _):
        vec = idx_all[pl.ds(pl.multiple_of(v * 16, 8), 16)]
        m = jnp.logical_and(vec >= lo_r, vec < hi_r)
        crel = lax.shift_right_logical(vec - lo_r, CHUNK_SHIFT)
        cnt = plsc.all_reduce_population_count(m)[0]

        def one_hit(_, mb):
            lane = plsc.all_reduce_ffs(mb)
            crel_j = jnp.take_along_axis(
                crel, lane, axis=0,
                mode=lax.GatherScatterMode.PROMISE_IN_BOUNDS)
            r_j = jnp.take_along_axis(
                vec, lane, axis=0,
                mode=lax.GatherScatterMode.PROMISE_IN_BOUNDS)
            slot = plsc.load_gather(cur_v, [crel_j])
            lane0 = lanes == 0
            plsc.store_scatter(hits_r, [slot], r_j, mask=lane0)
            plsc.store_scatter(hits_k, [slot], v * 16 + lane, mask=lane0)
            plsc.addupdate_scatter(cur_v, [crel_j], ones, mask=lane0)
            return jnp.logical_and(mb, lanes != lane)

        lax.fori_loop(0, cnt, one_hit, m)
        return _

    lax.fori_loop(0, NVREG, place, None)

    def window(c):
        lo = jnp.minimum((base_c + c) * CHUNK, MAX_LANE_LO)
        return pl.multiple_of(lo, 128)

    sem_chunk = (sem_ch0, sem_ch1, sem_ch2, sem_ch3)
    for b in range(NBUF):
        @pl.when(b < n_c)
        def _prime():
            pltpu.async_copy(tableT_hbm.at[:, pl.ds(window(b), CHUNK)],
                             chunk_v.at[b], sem_chunk[b])

    def process_chunk(c, bufi, lane_lo):
        cvec = jnp.full((16,), c, jnp.int32)
        bvec = jnp.full((16,), bufi, jnp.int32)
        s = plsc.load_gather(csr_v, [cvec])[0]
        e = plsc.load_gather(csr_v, [cvec + 1])[0]

        def batch(t, _):
            b0 = s + t * RING
            nb = jnp.minimum(e - b0, RING)

            def one(i, _):
                hvec = jnp.full((16,), b0 + i, jnp.int32)
                r = plsc.load_gather(hits_r, [hvec])[0]
                k = plsc.load_gather(hits_k, [hvec])[0]
                rl = jnp.full((16,), r - lane_lo, jnp.int32)
                for fb in range(4):
                    col = plsc.load_gather(
                        chunk_v, [bvec, fb * 16 + lanes, rl])
                    rowbuf[i, pl.ds(fb * 16, 16)] = col
                pltpu.async_copy(rowbuf.at[pl.ds(i, 1)],
                                 out_rows_hbm.at[pl.ds(k, 1)], sem_rows)
                return _

            lax.fori_loop(0, nb, one, None)

            def drain(i, _):
                pltpu.make_async_copy(rowbuf.at[pl.ds(0, 1)],
                                      out_rows_hbm.at[pl.ds(0, 1)],
                                      sem_rows).wait()
                return _

            lax.fori_loop(0, nb, drain, None)
            return _

        n_batches = lax.div(e - s + (RING - 1), RING)
        lax.fori_loop(0, n_batches, batch, None)

    def quad(p, _):
        for b in range(NBUF):
            c = p * NBUF + b

            @pl.when(c < n_c)
            def _do():
                pltpu.make_async_copy(
                    tableT_hbm.at[:, pl.ds(0, CHUNK)], chunk_v.at[b],
                    sem_chunk[b]).wait()
                process_chunk(c, b, window(c))

                @pl.when(c + NBUF < n_c)
                def _prefetch():
                    pltpu.async_copy(
                        tableT_hbm.at[:, pl.ds(window(c + NBUF), CHUNK)],
                        chunk_v.at[b], sem_chunk[b])
        return _

    lax.fori_loop(0, lax.div(n_c + (NBUF - 1), NBUF), quad, None)

    pltpu.make_async_copy(resp_hbm.at[pl.ds(0, 16 * B_PER_W)], resp16_v,
                          sem_resp).wait()

    def pick(g, _):
        vec = idx_all[pl.ds(pl.multiple_of(base + g * 16, 8), 16)]
        sub = lax.bitwise_and(vec, 15)
        flat = (g * 16 + lanes) * 16 + sub
        resp_v[pl.ds(pl.multiple_of(g * 16, 8), 16)] = (
            plsc.load_gather(resp16_v, [flat]))
        return _

    lax.fori_loop(0, NG, pick, None)
    pltpu.sync_copy(resp_v, out_resp_hbm.at[pl.ds(base, B_PER_W)])


def kernel(input, response, mb_idx):
    mesh = plsc.VectorSubcoreMesh(core_axis_name="c", subcore_axis_name="s")
    out_rows, out_resp = pl.kernel(
        _gather_kernel,
        mesh=mesh,
        compiler_params=pltpu.CompilerParams(needs_layout_passes=False),
        out_type=(
            jax.ShapeDtypeStruct((BATCH, D_FEAT), jnp.float32),
            jax.ShapeDtypeStruct((BATCH,), jnp.float32),
        ),
        scratch_types=[
            pltpu.VMEM((BATCH,), jnp.int32),
            pltpu.VMEM((160,), jnp.int32),
            pltpu.VMEM((160,), jnp.int32),
            pltpu.VMEM((160,), jnp.int32),
            pltpu.VMEM((BATCH,), jnp.int32),
            pltpu.VMEM((BATCH,), jnp.int32),
            pltpu.VMEM((NBUF, D_FEAT, CHUNK), jnp.float32),
            pltpu.VMEM((RING, D_FEAT), jnp.float32),
            pltpu.VMEM((16 * B_PER_W,), jnp.float32),
            pltpu.VMEM((B_PER_W,), jnp.float32),
            pltpu.SemaphoreType.DMA,
            pltpu.SemaphoreType.DMA,
            pltpu.SemaphoreType.DMA,
            pltpu.SemaphoreType.DMA,
            pltpu.SemaphoreType.DMA,
            pltpu.SemaphoreType.DMA,
        ],
    )(input.T, response, mb_idx)
    return out_rows, out_resp


# speedup vs baseline: 3.9903x; 1.0440x over previous
import jax
import jax.numpy as jnp
from jax import lax
from jax.experimental import pallas as pl
from jax.experimental.pallas import tpu as pltpu
from jax.experimental.pallas import tpu_sc as plsc

N_ROWS = 1000000
D_FEAT = 64
BATCH = 16384

NC = 2
NS = 16
NW = NC * NS
B_PER_W = BATCH // NW
NG = B_PER_W // 16
NVREG = BATCH // 16

NBUF = 8
CHUNK = 128
CHUNK_SHIFT = 7
N_CHUNKS = (N_ROWS + CHUNK - 1) // CHUNK
PHYS_MINOR = ((N_ROWS + 127) // 128) * 128
MAX_LANE_LO = ((PHYS_MINOR - CHUNK) // 128) * 128
CH_BASE = N_CHUNKS // NW
CH_EXTRA = N_CHUNKS - CH_BASE * NW
RING = 32


def _gather_kernel(tableT_hbm, resp_hbm, idx_hbm, out_rows_hbm, out_resp_hbm,
                   idx_all, hist_v, csr_v, cur_v, hits_r, hits_k,
                   chunk_v, rowbuf, resp16_v, resp_v,
                   sem_rows, sem_resp, sem_ch0, sem_ch1, sem_ch2, sem_ch3, sem_ch4, sem_ch5, sem_ch6, sem_ch7):
    wid = lax.axis_index("s") * NC + lax.axis_index("c")
    base = wid * B_PER_W
    lanes = lax.iota(jnp.int32, 16)
    ones = jnp.ones((16,), jnp.int32)

    pltpu.sync_copy(idx_hbm.at[pl.ds(0, BATCH)], idx_all)

    def fire_resp(g, _):
        vec = idx_all[pl.ds(pl.multiple_of(base + g * 16, 8), 16)]
        for j in range(16):
            k = g * 16 + j
            r16 = pl.multiple_of(
                lax.shift_left(lax.shift_right_logical(vec[j], 4), 4), 8)
            pltpu.async_copy(resp_hbm.at[pl.ds(r16, 16)],
                             resp16_v.at[pl.ds(pl.multiple_of(k * 16, 8), 16)],
                             sem_resp)
        return _

    lax.fori_loop(0, NG, fire_resp, None)

    n_c = CH_BASE + jnp.where(wid < CH_EXTRA, 1, 0)
    base_c = wid * CH_BASE + jnp.minimum(wid, CH_EXTRA)
    lo_r = base_c * CHUNK
    hi_r = (base_c + n_c) * CHUNK

    for i in range(16):
        hist_v[pl.ds(i * 16, 16)] = jnp.zeros((16,), jnp.int32)

    def hist(v, _):
        vec = idx_all[pl.ds(pl.multiple_of(v * 16, 8), 16)]
        m = jnp.logical_and(vec >= lo_r, vec < hi_r)
        crel = lax.shift_right_logical(vec - lo_r, CHUNK_SHIFT)
        plsc.addupdate_scatter(hist_v, [crel], ones, mask=m)
        return _

    lax.fori_loop(0, NVREG, hist, None)

    carry = jnp.zeros((16,), jnp.int32)
    for i in range(16):
        h = hist_v[pl.ds(i * 16, 16)]
        inc = plsc.cumsum(h)
        csr_v[pl.ds(i * 16, 16)] = inc - h + carry
        cur_v[pl.ds(i * 16, 16)] = inc - h + carry
        tot = jnp.take_along_axis(
            inc, jnp.full((16,), 15, jnp.int32), axis=0,
            mode=lax.GatherScatterMode.PROMISE_IN_BOUNDS)
        carry = carry + tot
    csr_v[pl.ds(256, 16)] = carry

    def place(v, _):
        vec = idx_all[pl.ds(pl.multiple_of(v * 16, 8), 16)]
        m = jnp.logical_and(vec >= lo_r, vec < hi_r)
        crel = lax.shift_right_logical(vec - lo_r, CHUNK_SHIFT)
        cnt = plsc.all_reduce_population_count(m)[0]

        def one_hit(_, mb):
            lane = plsc.all_reduce_ffs(mb)
            crel_j = jnp.take_along_axis(
                crel, lane, axis=0,
                mode=lax.GatherScatterMode.PROMISE_IN_BOUNDS)
            r_j = jnp.take_along_axis(
                vec, lane, axis=0,
                mode=lax.GatherScatterMode.PROMISE_IN_BOUNDS)
            slot = plsc.load_gather(cur_v, [crel_j])
            lane0 = lanes == 0
            plsc.store_scatter(hits_r, [slot], r_j, mask=lane0)
            plsc.store_scatter(hits_k, [slot], v * 16 + lane, mask=lane0)
            plsc.addupdate_scatter(cur_v, [crel_j], ones, mask=lane0)
            return jnp.logical_and(mb, lanes != lane)

        lax.fori_loop(0, cnt, one_hit, m)
        return _

    lax.fori_loop(0, NVREG, place, None)

    def window(c):
        lo = jnp.minimum((base_c + c) * CHUNK, MAX_LANE_LO)
        return pl.multiple_of(lo, 128)

    sem_chunk = (sem_ch0, sem_ch1, sem_ch2, sem_ch3, sem_ch4, sem_ch5, sem_ch6, sem_ch7)
    for b in range(NBUF):
        @pl.when(b < n_c)
        def _prime():
            pltpu.async_copy(tableT_hbm.at[:, pl.ds(window(b), CHUNK)],
                             chunk_v.at[b], sem_chunk[b])

    def process_chunk(c, bufi, lane_lo):
        cvec = jnp.full((16,), c, jnp.int32)
        bvec = jnp.full((16,), bufi, jnp.int32)
        s = plsc.load_gather(csr_v, [cvec])[0]
        e = plsc.load_gather(csr_v, [cvec + 1])[0]

        def batch(t, _):
            b0 = s + t * RING
            nb = jnp.minimum(e - b0, RING)

            def one(i, _):
                hvec = jnp.full((16,), b0 + i, jnp.int32)
                r = plsc.load_gather(hits_r, [hvec])[0]
                k = plsc.load_gather(hits_k, [hvec])[0]
                rl = jnp.full((16,), r - lane_lo, jnp.int32)
                for fb in range(4):
                    col = plsc.load_gather(
                        chunk_v, [bvec, fb * 16 + lanes, rl])
                    rowbuf[i, pl.ds(fb * 16, 16)] = col
                pltpu.async_copy(rowbuf.at[pl.ds(i, 1)],
                                 out_rows_hbm.at[pl.ds(k, 1)], sem_rows)
                return _

            lax.fori_loop(0, nb, one, None)

            def drain(i, _):
                pltpu.make_async_copy(rowbuf.at[pl.ds(0, 1)],
                                      out_rows_hbm.at[pl.ds(0, 1)],
                                      sem_rows).wait()
                return _

            lax.fori_loop(0, nb, drain, None)
            return _

        n_batches = lax.div(e - s + (RING - 1), RING)
        lax.fori_loop(0, n_batches, batch, None)

    def quad(p, _):
        for b in range(NBUF):
            c = p * NBUF + b

            @pl.when(c < n_c)
            def _do():
                pltpu.make_async_copy(
                    tableT_hbm.at[:, pl.ds(0, CHUNK)], chunk_v.at[b],
                    sem_chunk[b]).wait()
                process_chunk(c, b, window(c))

                @pl.when(c + NBUF < n_c)
                def _prefetch():
                    pltpu.async_copy(
                        tableT_hbm.at[:, pl.ds(window(c + NBUF), CHUNK)],
                        chunk_v.at[b], sem_chunk[b])
        return _

    lax.fori_loop(0, lax.div(n_c + (NBUF - 1), NBUF), quad, None)

    pltpu.make_async_copy(resp_hbm.at[pl.ds(0, 16 * B_PER_W)], resp16_v,
                          sem_resp).wait()

    def pick(g, _):
        vec = idx_all[pl.ds(pl.multiple_of(base + g * 16, 8), 16)]
        sub = lax.bitwise_and(vec, 15)
        flat = (g * 16 + lanes) * 16 + sub
        resp_v[pl.ds(pl.multiple_of(g * 16, 8), 16)] = (
            plsc.load_gather(resp16_v, [flat]))
        return _

    lax.fori_loop(0, NG, pick, None)
    pltpu.sync_copy(resp_v, out_resp_hbm.at[pl.ds(base, B_PER_W)])


def kernel(input, response, mb_idx):
    mesh = plsc.VectorSubcoreMesh(core_axis_name="c", subcore_axis_name="s")
    out_rows, out_resp = pl.kernel(
        _gather_kernel,
        mesh=mesh,
        compiler_params=pltpu.CompilerParams(needs_layout_passes=False),
        out_type=(
            jax.ShapeDtypeStruct((BATCH, D_FEAT), jnp.float32),
            jax.ShapeDtypeStruct((BATCH,), jnp.float32),
        ),
        scratch_types=[
            pltpu.VMEM((BATCH,), jnp.int32),
            pltpu.VMEM((272,), jnp.int32),
            pltpu.VMEM((272,), jnp.int32),
            pltpu.VMEM((272,), jnp.int32),
            pltpu.VMEM((BATCH,), jnp.int32),
            pltpu.VMEM((BATCH,), jnp.int32),
            pltpu.VMEM((NBUF, D_FEAT, CHUNK), jnp.float32),
            pltpu.VMEM((RING, D_FEAT), jnp.float32),
            pltpu.VMEM((16 * B_PER_W,), jnp.float32),
            pltpu.VMEM((B_PER_W,), jnp.float32),
            pltpu.SemaphoreType.DMA,
            pltpu.SemaphoreType.DMA,
            pltpu.SemaphoreType.DMA,
            pltpu.SemaphoreType.DMA,
            pltpu.SemaphoreType.DMA,
            pltpu.SemaphoreType.DMA,
            pltpu.SemaphoreType.DMA,
            pltpu.SemaphoreType.DMA,
            pltpu.SemaphoreType.DMA,
            pltpu.SemaphoreType.DMA,
        ],
    )(input.T, response, mb_idx)
    return out_rows, out_resp


# speedup vs baseline: 4.0054x; 1.0038x over previous
import jax
import jax.numpy as jnp
from jax import lax
from jax.experimental import pallas as pl
from jax.experimental.pallas import tpu as pltpu
from jax.experimental.pallas import tpu_sc as plsc

N_ROWS = 1000000
D_FEAT = 64
BATCH = 16384

NC = 2
NS = 16
NW = NC * NS
B_PER_W = BATCH // NW
NG = B_PER_W // 16
NVREG = BATCH // 16

NBUF = 8
CHUNK = 128
CHUNK_SHIFT = 7
N_CHUNKS = (N_ROWS + CHUNK - 1) // CHUNK
PHYS_MINOR = ((N_ROWS + 127) // 128) * 128
MAX_LANE_LO = ((PHYS_MINOR - CHUNK) // 128) * 128
CH_BASE = N_CHUNKS // NW
CH_EXTRA = N_CHUNKS - CH_BASE * NW
RING = 32


def _gather_kernel(tableT_hbm, resp_hbm, idx_hbm, out_rows_hbm, out_resp_hbm,
                   idx_all, hist_v, csr_v, cur_v, hits_r, hits_k,
                   chunk_v, rowbuf, resp16_v, resp_v,
                   sem_rows, sem_resp, sem_ch0, sem_ch1, sem_ch2, sem_ch3, sem_ch4, sem_ch5, sem_ch6, sem_ch7):
    wid = lax.axis_index("s") * NC + lax.axis_index("c")
    base = wid * B_PER_W
    lanes = lax.iota(jnp.int32, 16)
    ones = jnp.ones((16,), jnp.int32)

    pltpu.sync_copy(idx_hbm.at[pl.ds(0, BATCH)], idx_all)

    n_c = CH_BASE + jnp.where(wid < CH_EXTRA, 1, 0)
    base_c = wid * CH_BASE + jnp.minimum(wid, CH_EXTRA)
    lo_r = base_c * CHUNK
    hi_r = (base_c + n_c) * CHUNK

    def window(c):
        lo = jnp.minimum((base_c + c) * CHUNK, MAX_LANE_LO)
        return pl.multiple_of(lo, 128)

    sem_chunk = (sem_ch0, sem_ch1, sem_ch2, sem_ch3, sem_ch4, sem_ch5, sem_ch6, sem_ch7)
    for b in range(NBUF):
        @pl.when(b < n_c)
        def _prime():
            pltpu.async_copy(tableT_hbm.at[:, pl.ds(window(b), CHUNK)],
                             chunk_v.at[b], sem_chunk[b])


    def fire_resp(g, _):
        vec = idx_all[pl.ds(pl.multiple_of(base + g * 16, 8), 16)]
        for j in range(16):
            k = g * 16 + j
            r16 = pl.multiple_of(
                lax.shift_left(lax.shift_right_logical(vec[j], 4), 4), 8)
            pltpu.async_copy(resp_hbm.at[pl.ds(r16, 16)],
                             resp16_v.at[pl.ds(pl.multiple_of(k * 16, 8), 16)],
                             sem_resp)
        return _

    lax.fori_loop(0, NG, fire_resp, None)

    for i in range(16):
        hist_v[pl.ds(i * 16, 16)] = jnp.zeros((16,), jnp.int32)

    def hist(v, _):
        vec = idx_all[pl.ds(pl.multiple_of(v * 16, 8), 16)]
        m = jnp.logical_and(vec >= lo_r, vec < hi_r)
        crel = lax.shift_right_logical(vec - lo_r, CHUNK_SHIFT)
        plsc.addupdate_scatter(hist_v, [crel], ones, mask=m)
        return _

    lax.fori_loop(0, NVREG, hist, None)

    carry = jnp.zeros((16,), jnp.int32)
    for i in range(16):
        h = hist_v[pl.ds(i * 16, 16)]
        inc = plsc.cumsum(h)
        csr_v[pl.ds(i * 16, 16)] = inc - h + carry
        cur_v[pl.ds(i * 16, 16)] = inc - h + carry
        tot = jnp.take_along_axis(
            inc, jnp.full((16,), 15, jnp.int32), axis=0,
            mode=lax.GatherScatterMode.PROMISE_IN_BOUNDS)
        carry = carry + tot
    csr_v[pl.ds(256, 16)] = carry

    def place(v, _):
        vec = idx_all[pl.ds(pl.multiple_of(v * 16, 8), 16)]
        m = jnp.logical_and(vec >= lo_r, vec < hi_r)
        crel = lax.shift_right_logical(vec - lo_r, CHUNK_SHIFT)
        cnt = plsc.all_reduce_population_count(m)[0]

        def one_hit(_, mb):
            lane = plsc.all_reduce_ffs(mb)
            crel_j = jnp.take_along_axis(
                crel, lane, axis=0,
                mode=lax.GatherScatterMode.PROMISE_IN_BOUNDS)
            r_j = jnp.take_along_axis(
                vec, lane, axis=0,
                mode=lax.GatherScatterMode.PROMISE_IN_BOUNDS)
            slot = plsc.load_gather(cur_v, [crel_j])
            lane0 = lanes == 0
            plsc.store_scatter(hits_r, [slot], r_j, mask=lane0)
            plsc.store_scatter(hits_k, [slot], v * 16 + lane, mask=lane0)
            plsc.addupdate_scatter(cur_v, [crel_j], ones, mask=lane0)
            return jnp.logical_and(mb, lanes != lane)

        lax.fori_loop(0, cnt, one_hit, m)
        return _

    lax.fori_loop(0, NVREG, place, None)

    def process_chunk(c, bufi, lane_lo):
        cvec = jnp.full((16,), c, jnp.int32)
        bvec = jnp.full((16,), bufi, jnp.int32)
        s = plsc.load_gather(csr_v, [cvec])[0]
        e = plsc.load_gather(csr_v, [cvec + 1])[0]

        def batch(t, _):
            b0 = s + t * RING
            nb = jnp.minimum(e - b0, RING)

            def one(i, _):
                hvec = jnp.full((16,), b0 + i, jnp.int32)
                r = plsc.load_gather(hits_r, [hvec])[0]
                k = plsc.load_gather(hits_k, [hvec])[0]
                rl = jnp.full((16,), r - lane_lo, jnp.int32)
                for fb in range(4):
                    col = plsc.load_gather(
                        chunk_v, [bvec, fb * 16 + lanes, rl])
                    rowbuf[i, pl.ds(fb * 16, 16)] = col
                pltpu.async_copy(rowbuf.at[pl.ds(i, 1)],
                                 out_rows_hbm.at[pl.ds(k, 1)], sem_rows)
                return _

            lax.fori_loop(0, nb, one, None)

            def drain(i, _):
                pltpu.make_async_copy(rowbuf.at[pl.ds(0, 1)],
                                      out_rows_hbm.at[pl.ds(0, 1)],
                                      sem_rows).wait()
                return _

            lax.fori_loop(0, nb, drain, None)
            return _

        n_batches = lax.div(e - s + (RING - 1), RING)
        lax.fori_loop(0, n_batches, batch, None)

    def quad(p, _):
        for b in range(NBUF):
            c = p * NBUF + b

            @pl.when(c < n_c)
            def _do():
                pltpu.make_async_copy(
                    tableT_hbm.at[:, pl.ds(0, CHUNK)], chunk_v.at[b],
                    sem_chunk[b]).wait()
                process_chunk(c, b, window(c))

                @pl.when(c + NBUF < n_c)
                def _prefetch():
                    pltpu.async_copy(
                        tableT_hbm.at[:, pl.ds(window(c + NBUF), CHUNK)],
                        chunk_v.at[b], sem_chunk[b])
        return _

    lax.fori_loop(0, lax.div(n_c + (NBUF - 1), NBUF), quad, None)

    pltpu.make_async_copy(resp_hbm.at[pl.ds(0, 16 * B_PER_W)], resp16_v,
                          sem_resp).wait()

    def pick(g, _):
        vec = idx_all[pl.ds(pl.multiple_of(base + g * 16, 8), 16)]
        sub = lax.bitwise_and(vec, 15)
        flat = (g * 16 + lanes) * 16 + sub
        resp_v[pl.ds(pl.multiple_of(g * 16, 8), 16)] = (
            plsc.load_gather(resp16_v, [flat]))
        return _

    lax.fori_loop(0, NG, pick, None)
    pltpu.sync_copy(resp_v, out_resp_hbm.at[pl.ds(base, B_PER_W)])


def kernel(input, response, mb_idx):
    mesh = plsc.VectorSubcoreMesh(core_axis_name="c", subcore_axis_name="s")
    out_rows, out_resp = pl.kernel(
        _gather_kernel,
        mesh=mesh,
        compiler_params=pltpu.CompilerParams(needs_layout_passes=False),
        out_type=(
            jax.ShapeDtypeStruct((BATCH, D_FEAT), jnp.float32),
            jax.ShapeDtypeStruct((BATCH,), jnp.float32),
        ),
        scratch_types=[
            pltpu.VMEM((BATCH,), jnp.int32),
            pltpu.VMEM((272,), jnp.int32),
            pltpu.VMEM((272,), jnp.int32),
            pltpu.VMEM((272,), jnp.int32),
            pltpu.VMEM((BATCH,), jnp.int32),
            pltpu.VMEM((BATCH,), jnp.int32),
            pltpu.VMEM((NBUF, D_FEAT, CHUNK), jnp.float32),
            pltpu.VMEM((RING, D_FEAT), jnp.float32),
            pltpu.VMEM((16 * B_PER_W,), jnp.float32),
            pltpu.VMEM((B_PER_W,), jnp.float32),
            pltpu.SemaphoreType.DMA,
            pltpu.SemaphoreType.DMA,
            pltpu.SemaphoreType.DMA,
            pltpu.SemaphoreType.DMA,
            pltpu.SemaphoreType.DMA,
            pltpu.SemaphoreType.DMA,
            pltpu.SemaphoreType.DMA,
            pltpu.SemaphoreType.DMA,
            pltpu.SemaphoreType.DMA,
            pltpu.SemaphoreType.DMA,
        ],
    )(input.T, response, mb_idx)
    return out_rows, out_resp
